# baseline (speedup 1.0000x reference)
import dataclasses
import functools

import jax
import jax.numpy as jnp
import numpy as np
from jax.experimental import pallas as pl
from jax.experimental.pallas import tpu as pltpu
from jax.experimental.pallas import tpu_sc as plsc

PERIOD = 26
ROWS = 32
SC_ROWS = 32


def _aux_constants(S: int):
    out_len = S + 1
    scale = out_len / S
    i = np.arange(S, dtype=np.float32)
    src = np.clip((i + np.float32(0.5)) * np.float32(scale) - np.float32(0.5),
                  np.float32(0.0), np.float32(out_len - 1)).astype(np.float32)
    w = (src - i).astype(np.float32)
    ii = np.arange(S)
    nneg = np.maximum(0, 12 - ii).astype(np.float32)
    npos = np.maximum(0, ii - (S - 13)).astype(np.float32)
    alpha = nneg + (1.0 - w) * (ii <= 12).astype(np.float32)
    beta = npos + w * (ii >= S - 13).astype(np.float32)
    aux = np.stack([w, alpha, beta])
    ph = (np.arange(S) % PERIOD)
    H = np.zeros((S, 32), dtype=np.float32)
    H[np.arange(S), ph] = 1.0
    Ht = np.zeros((32, S), dtype=np.float32)
    Ht[ph, np.arange(S)] = 1.0
    return aux, H.astype(jnp.bfloat16), Ht.astype(jnp.bfloat16)


def _body(S, x_ref, w_ref, al_ref, be_ref, h_ref, ht_ref,
          tr_ref, se_ref, re_ref, pa, pb, pc):
    R = ROWS
    KR = x_ref.shape[0]
    W = S + 256

    @pl.when(pl.program_id(0) == 0)
    def _():
        z128 = jnp.zeros((R, 128), jnp.float32)
        pa[:, 0:128] = z128
        pa[:, S + 128:S + 384] = jnp.zeros((R, 256), jnp.float32)
        pb[:, S + 256:S + 384] = z128
        pc[:, S + 256:S + 384] = z128

    X = jnp.reshape(x_ref[...], (R, S))

    pa[:, 128:128 + S] = X

    def rd_a(d):
        return pa[:, d:d + W]

    def rd(buf, d):
        return buf[:, d:d + W]

    def wr(buf, v):
        buf[:, 0:W] = v

    wr(pb, rd_a(0) + rd_a(1))
    wr(pc, rd(pb, 0) + rd_a(2))
    wr(pb, rd(pc, 0) + rd(pc, 3))
    wr(pc, rd(pb, 0) + rd(pb, 6))
    wr(pb, rd(pc, 0) + rd(pc, 12))

    S_z = pb[:, 116:116 + S] + pa[:, 140:140 + S]
    xm13 = pa[:, 115:115 + S]
    xp13 = pa[:, 141:141 + S]

    w = w_ref[...]
    al = al_ref[...]
    be = be_ref[...]
    x0 = X[:, 0:1]
    xL = X[:, S - 1:S]
    trend = (S_z + (1.0 - w) * xm13 + w * xp13 + al * x0 + be * xL) * (
        np.float32(1.0 / PERIOD))
    D = X - trend

    Dhi = D.astype(jnp.bfloat16)
    Dlo = (D - Dhi.astype(jnp.float32)).astype(jnp.bfloat16)
    H = h_ref[...]
    sums = (jnp.dot(Dhi, H, preferred_element_type=jnp.float32)
            + jnp.dot(Dlo, H, preferred_element_type=jnp.float32))
    p = jax.lax.broadcasted_iota(jnp.int32, (R, 32), 1)
    counts = ((S - 1 - p) // PERIOD + 1).astype(jnp.float32)
    means = sums / counts
    mhi = means.astype(jnp.bfloat16)
    mlo = (means - mhi.astype(jnp.float32)).astype(jnp.bfloat16)
    Ht = ht_ref[...]
    seasonal = (jnp.dot(mhi, Ht, preferred_element_type=jnp.float32)
                + jnp.dot(mlo, Ht, preferred_element_type=jnp.float32))

    tr_ref[...] = jnp.reshape(trend, (KR, 128))
    se_ref[...] = jnp.reshape(seasonal, (KR, 128))
    re_ref[...] = jnp.reshape(D - seasonal, (KR, 128))


@functools.lru_cache(maxsize=4)
def _make_tc(B, S, btc=None, interpret=False):
    aux_np, H_np, Ht_np = _aux_constants(S)
    R = ROWS
    w_c = jnp.asarray(np.broadcast_to(aux_np[0], (R, S)).copy())
    al_c = jnp.asarray(np.broadcast_to(aux_np[1], (R, S)).copy())
    be_c = jnp.asarray(np.broadcast_to(aux_np[2], (R, S)).copy())
    H_c = jnp.asarray(H_np)
    Ht_c = jnp.asarray(Ht_np)
    N = (B if btc is None else btc) // R
    M = B * S // 128
    out_sd = jax.ShapeDtypeStruct((M, 128), jnp.float32)
    io_spec = pl.BlockSpec((R * S // 128, 128), lambda i: (i, 0))
    call = pl.pallas_call(
        functools.partial(_body, S),
        grid=(N,),
        in_specs=[
            io_spec,
            pl.BlockSpec((R, S), lambda i: (0, 0)),
            pl.BlockSpec((R, S), lambda i: (0, 0)),
            pl.BlockSpec((R, S), lambda i: (0, 0)),
            pl.BlockSpec((S, 32), lambda i: (0, 0)),
            pl.BlockSpec((32, S), lambda i: (0, 0)),
        ],
        out_specs=[io_spec, io_spec, io_spec],
        out_shape=[out_sd, out_sd, out_sd],
        scratch_shapes=[
            pltpu.VMEM((R, S + 384), jnp.float32),
            pltpu.VMEM((R, S + 384), jnp.float32),
            pltpu.VMEM((R, S + 384), jnp.float32),
        ],
        interpret=interpret,
    )

    def run(x):
        return call(x, w_c, al_c, be_c, H_c, Ht_c)

    return run


def _sc_body(S, row0, x_hbm, tr_hbm, se_hbm, re_hbm,
             xp, cbuf, dbuf, trbuf, sebuf, rebuf, acc):
    L = 16
    WPAD = S + 48
    ncore = jax.lax.axis_index("c")
    nsub = jax.lax.axis_index("s")
    wid = nsub * 2 + ncore
    row = row0 + wid
    inv26 = np.float32(1.0 / PERIOD)
    scale = np.float32((S + 1) / S)

    pltpu.sync_copy(x_hbm.at[pl.ds(row * S, S)], xp.at[pl.ds(16, S)])
    v0 = xp[pl.ds(16, L)]
    vL = xp[pl.ds(S, L)]
    x0 = jnp.full((L,), v0[0], jnp.float32)
    xL = jnp.full((L,), vL[15], jnp.float32)
    xp[pl.ds(0, L)] = x0
    xp[pl.ds(S + 16, L)] = xL
    xp[pl.ds(S + 32, L)] = xL

    def cumsum_step(k16, carry):
        k = k16 * L
        v = xp[pl.ds(k, L)]
        cbuf[pl.ds(k, L)] = plsc.cumsum(v) + carry
        return carry + jnp.sum(v)

    jax.lax.fori_loop(0, WPAD // L, cumsum_step, jnp.float32(0.0),
                      unroll=2)

    acc[pl.ds(0, L)] = jnp.zeros((L,), jnp.float32)
    acc[pl.ds(L, L)] = jnp.zeros((L,), jnp.float32)
    iota = jax.lax.iota(jnp.int32, L)

    def main_step(k16, carry):
        k = k16 * L
        ca = cbuf[pl.ds(k + 2, L)]
        cb = cbuf[pl.ds(k + 28, L)]
        ca1 = cbuf[pl.ds(k + 3, L)]
        cb1 = cbuf[pl.ds(k + 29, L)]
        y0 = (cb - ca) * inv26
        y1 = (cb1 - ca1) * inv26
        ivec = iota + k
        i_f = ivec.astype(jnp.float32)
        src = (i_f + np.float32(0.5)) * scale - np.float32(0.5)
        w = src - i_f
        t = y0 + w * (y1 - y0)
        xv = xp[pl.ds(k + 16, L)]
        d = xv - t
        trbuf[pl.ds(k, L)] = t
        dbuf[pl.ds(k, L)] = d
        ph = jax.lax.rem(ivec, PERIOD)
        plsc.addupdate_scatter(acc, [ph], d)
        return carry

    jax.lax.fori_loop(0, S // L, main_step, jnp.int32(0), unroll=2)

    for c in range(2):
        p = iota + c * L
        cnt = ((S - 1 - p) // PERIOD + 1).astype(jnp.float32)
        acc[pl.ds(c * L, L)] = acc[pl.ds(c * L, L)] / cnt

    def bcast_step(k16, carry):
        k = k16 * L
        ph = jax.lax.rem(iota + k, PERIOD)
        se = plsc.load_gather(acc, [ph])
        sebuf[pl.ds(k, L)] = se
        rebuf[pl.ds(k, L)] = dbuf[pl.ds(k, L)] - se
        return carry

    jax.lax.fori_loop(0, S // L, bcast_step, jnp.int32(0), unroll=2)

    pltpu.sync_copy(trbuf, tr_hbm.at[pl.ds(wid * S, S)])
    pltpu.sync_copy(sebuf, se_hbm.at[pl.ds(wid * S, S)])
    pltpu.sync_copy(rebuf, re_hbm.at[pl.ds(wid * S, S)])


@functools.lru_cache(maxsize=2)
def _make_sc(B, S, row0):
    WPAD = S + 48
    mesh = plsc.VectorSubcoreMesh(core_axis_name="c", subcore_axis_name="s")
    out_sd = jax.ShapeDtypeStruct((SC_ROWS * S,), jnp.float32)
    cp = pltpu.CompilerParams()
    if "needs_layout_passes" in pltpu.CompilerParams.__dataclass_fields__:
        cp = dataclasses.replace(cp, needs_layout_passes=False)
    return pl.kernel(
        functools.partial(_sc_body, S, row0),
        out_type=[out_sd, out_sd, out_sd],
        mesh=mesh,
        compiler_params=cp,
        scratch_types=[
            pltpu.VMEM((WPAD,), jnp.float32),
            pltpu.VMEM((WPAD,), jnp.float32),
            pltpu.VMEM((S,), jnp.float32),
            pltpu.VMEM((S,), jnp.float32),
            pltpu.VMEM((S,), jnp.float32),
            pltpu.VMEM((S,), jnp.float32),
            pltpu.VMEM((2 * 16,), jnp.float32),
        ],
    )


def kernel(x):
    B, S, _ = x.shape
    btc = B - SC_ROWS
    x2 = jnp.reshape(x, (B * S // 128, 128))
    trend, seasonal, resid = _make_tc(B, S, btc)(x2)
    x1 = jnp.reshape(x, (B * S,))
    sc_t, sc_s, sc_r = _make_sc(B, S, btc)(x1)

    shp = (B, S, 1)

    def merge(tc2d, sc1d):
        flat = jnp.reshape(tc2d, (B * S,))
        flat = jax.lax.dynamic_update_slice(flat, sc1d, (btc * S,))
        return jnp.reshape(flat, shp)

    return (merge(trend, sc_t), merge(seasonal, sc_s), merge(resid, sc_r))


# speedup vs baseline: 1.0139x; 1.0139x over previous
import dataclasses
import functools

import jax
import jax.numpy as jnp
import numpy as np
from jax.experimental import pallas as pl
from jax.experimental.pallas import tpu as pltpu
from jax.experimental.pallas import tpu_sc as plsc

PERIOD = 26
ROWS = 32
SC_ROWS = 32


def _aux_constants(S: int):
    out_len = S + 1
    scale = out_len / S
    i = np.arange(S, dtype=np.float32)
    src = np.clip((i + np.float32(0.5)) * np.float32(scale) - np.float32(0.5),
                  np.float32(0.0), np.float32(out_len - 1)).astype(np.float32)
    w = (src - i).astype(np.float32)
    ii = np.arange(S)
    nneg = np.maximum(0, 12 - ii).astype(np.float32)
    npos = np.maximum(0, ii - (S - 13)).astype(np.float32)
    alpha = nneg + (1.0 - w) * (ii <= 12).astype(np.float32)
    beta = npos + w * (ii >= S - 13).astype(np.float32)
    aux = np.zeros((8, S), dtype=np.float32)
    aux[0] = w
    aux[1] = alpha
    aux[2] = beta
    ph = (np.arange(S) % PERIOD)
    H = np.zeros((S, 32), dtype=np.float32)
    H[np.arange(S), ph] = 1.0
    Ht = np.zeros((32, S), dtype=np.float32)
    Ht[ph, np.arange(S)] = 1.0
    return aux, H.astype(jnp.bfloat16), Ht.astype(jnp.bfloat16)


def _body(S, x_ref, aux_ref, h_ref, ht_ref,
          tr_ref, se_ref, re_ref, pa, pb, pc):
    R = ROWS
    KR = x_ref.shape[0]
    W = S + 256

    @pl.when(pl.program_id(0) == 0)
    def _():
        z128 = jnp.zeros((R, 128), jnp.float32)
        pa[:, 0:128] = z128
        pa[:, S + 128:S + 384] = jnp.zeros((R, 256), jnp.float32)
        pb[:, S + 256:S + 384] = z128
        pc[:, S + 256:S + 384] = z128

    X = jnp.reshape(x_ref[...], (R, S))

    pa[:, 128:128 + S] = X

    def rd_a(d):
        return pa[:, d:d + W]

    def rd(buf, d):
        return buf[:, d:d + W]

    def wr(buf, v):
        buf[:, 0:W] = v

    wr(pb, rd_a(0) + rd_a(1))
    wr(pc, rd(pb, 0) + rd_a(2))
    wr(pb, rd(pc, 0) + rd(pc, 3))
    wr(pc, rd(pb, 0) + rd(pb, 6))
    wr(pb, rd(pc, 0) + rd(pc, 12))

    S_z = pb[:, 116:116 + S] + pa[:, 140:140 + S]
    xm13 = pa[:, 115:115 + S]
    xp13 = pa[:, 141:141 + S]

    w = aux_ref[0:1, :]
    al = aux_ref[1:2, :]
    be = aux_ref[2:3, :]
    x0 = X[:, 0:1]
    xL = X[:, S - 1:S]
    trend = (S_z + (1.0 - w) * xm13 + w * xp13 + al * x0 + be * xL) * (
        np.float32(1.0 / PERIOD))
    D = X - trend

    Dhi = D.astype(jnp.bfloat16)
    Dlo = (D - Dhi.astype(jnp.float32)).astype(jnp.bfloat16)
    H = h_ref[...]
    sums = (jnp.dot(Dhi, H, preferred_element_type=jnp.float32)
            + jnp.dot(Dlo, H, preferred_element_type=jnp.float32))
    p = jax.lax.broadcasted_iota(jnp.int32, (R, 32), 1)
    counts = ((S - 1 - p) // PERIOD + 1).astype(jnp.float32)
    means = sums / counts
    mhi = means.astype(jnp.bfloat16)
    mlo = (means - mhi.astype(jnp.float32)).astype(jnp.bfloat16)
    Ht = ht_ref[...]
    seasonal = (jnp.dot(mhi, Ht, preferred_element_type=jnp.float32)
                + jnp.dot(mlo, Ht, preferred_element_type=jnp.float32))

    tr_ref[...] = jnp.reshape(trend, (KR, 128))
    se_ref[...] = jnp.reshape(seasonal, (KR, 128))
    re_ref[...] = jnp.reshape(D - seasonal, (KR, 128))


@functools.lru_cache(maxsize=4)
def _make_tc(B, S, btc=None, interpret=False):
    aux_np, H_np, Ht_np = _aux_constants(S)
    R = ROWS
    aux_c = jnp.asarray(aux_np)
    H_c = jnp.asarray(H_np)
    Ht_c = jnp.asarray(Ht_np)
    N = (B if btc is None else btc) // R
    M = B * S // 128
    out_sd = jax.ShapeDtypeStruct((M, 128), jnp.float32)
    io_spec = pl.BlockSpec((R * S // 128, 128), lambda i: (i, 0))
    call = pl.pallas_call(
        functools.partial(_body, S),
        grid=(N,),
        in_specs=[
            io_spec,
            pl.BlockSpec((8, S), lambda i: (0, 0)),
            pl.BlockSpec((S, 32), lambda i: (0, 0)),
            pl.BlockSpec((32, S), lambda i: (0, 0)),
        ],
        out_specs=[io_spec, io_spec, io_spec],
        out_shape=[out_sd, out_sd, out_sd],
        scratch_shapes=[
            pltpu.VMEM((R, S + 384), jnp.float32),
            pltpu.VMEM((R, S + 384), jnp.float32),
            pltpu.VMEM((R, S + 384), jnp.float32),
        ],
        interpret=interpret,
    )

    def run(x):
        return call(x, aux_c, H_c, Ht_c)

    return run


def _sc_body(S, row0, x_hbm, tr_hbm, se_hbm, re_hbm,
             xp, cbuf, dbuf, trbuf, sebuf, rebuf, acc):
    L = 16
    WPAD = S + 48
    ncore = jax.lax.axis_index("c")
    nsub = jax.lax.axis_index("s")
    wid = nsub * 2 + ncore
    row = row0 + wid
    inv26 = np.float32(1.0 / PERIOD)
    scale = np.float32((S + 1) / S)

    pltpu.sync_copy(x_hbm.at[pl.ds(row * S, S)], xp.at[pl.ds(16, S)])
    v0 = xp[pl.ds(16, L)]
    vL = xp[pl.ds(S, L)]
    x0 = jnp.full((L,), v0[0], jnp.float32)
    xL = jnp.full((L,), vL[15], jnp.float32)
    xp[pl.ds(0, L)] = x0
    xp[pl.ds(S + 16, L)] = xL
    xp[pl.ds(S + 32, L)] = xL

    def cumsum_step(k16, carry):
        k = k16 * L
        v = xp[pl.ds(k, L)]
        cbuf[pl.ds(k, L)] = plsc.cumsum(v) + carry
        return carry + jnp.sum(v)

    jax.lax.fori_loop(0, WPAD // L, cumsum_step, jnp.float32(0.0),
                      unroll=2)

    acc[pl.ds(0, L)] = jnp.zeros((L,), jnp.float32)
    acc[pl.ds(L, L)] = jnp.zeros((L,), jnp.float32)
    iota = jax.lax.iota(jnp.int32, L)

    def main_step(k16, carry):
        k = k16 * L
        ca = cbuf[pl.ds(k + 2, L)]
        cb = cbuf[pl.ds(k + 28, L)]
        ca1 = cbuf[pl.ds(k + 3, L)]
        cb1 = cbuf[pl.ds(k + 29, L)]
        y0 = (cb - ca) * inv26
        y1 = (cb1 - ca1) * inv26
        ivec = iota + k
        i_f = ivec.astype(jnp.float32)
        src = (i_f + np.float32(0.5)) * scale - np.float32(0.5)
        w = src - i_f
        t = y0 + w * (y1 - y0)
        xv = xp[pl.ds(k + 16, L)]
        d = xv - t
        trbuf[pl.ds(k, L)] = t
        dbuf[pl.ds(k, L)] = d
        ph = jax.lax.rem(ivec, PERIOD)
        plsc.addupdate_scatter(acc, [ph], d)
        return carry

    jax.lax.fori_loop(0, S // L, main_step, jnp.int32(0), unroll=2)

    for c in range(2):
        p = iota + c * L
        cnt = ((S - 1 - p) // PERIOD + 1).astype(jnp.float32)
        acc[pl.ds(c * L, L)] = acc[pl.ds(c * L, L)] / cnt

    def bcast_step(k16, carry):
        k = k16 * L
        ph = jax.lax.rem(iota + k, PERIOD)
        se = plsc.load_gather(acc, [ph])
        sebuf[pl.ds(k, L)] = se
        rebuf[pl.ds(k, L)] = dbuf[pl.ds(k, L)] - se
        return carry

    jax.lax.fori_loop(0, S // L, bcast_step, jnp.int32(0), unroll=2)

    pltpu.sync_copy(trbuf, tr_hbm.at[pl.ds(wid * S, S)])
    pltpu.sync_copy(sebuf, se_hbm.at[pl.ds(wid * S, S)])
    pltpu.sync_copy(rebuf, re_hbm.at[pl.ds(wid * S, S)])


@functools.lru_cache(maxsize=2)
def _make_sc(B, S, row0):
    WPAD = S + 48
    mesh = plsc.VectorSubcoreMesh(core_axis_name="c", subcore_axis_name="s")
    out_sd = jax.ShapeDtypeStruct((SC_ROWS * S,), jnp.float32)
    cp = pltpu.CompilerParams()
    if "needs_layout_passes" in pltpu.CompilerParams.__dataclass_fields__:
        cp = dataclasses.replace(cp, needs_layout_passes=False)
    return pl.kernel(
        functools.partial(_sc_body, S, row0),
        out_type=[out_sd, out_sd, out_sd],
        mesh=mesh,
        compiler_params=cp,
        scratch_types=[
            pltpu.VMEM((WPAD,), jnp.float32),
            pltpu.VMEM((WPAD,), jnp.float32),
            pltpu.VMEM((S,), jnp.float32),
            pltpu.VMEM((S,), jnp.float32),
            pltpu.VMEM((S,), jnp.float32),
            pltpu.VMEM((S,), jnp.float32),
            pltpu.VMEM((2 * 16,), jnp.float32),
        ],
    )


def kernel(x):
    B, S, _ = x.shape
    btc = B - SC_ROWS
    x2 = jnp.reshape(x, (B * S // 128, 128))
    trend, seasonal, resid = _make_tc(B, S, btc)(x2)
    x1 = jnp.reshape(x, (B * S,))
    sc_t, sc_s, sc_r = _make_sc(B, S, btc)(x1)

    shp = (B, S, 1)

    def merge(tc2d, sc1d):
        flat = jnp.reshape(tc2d, (B * S,))
        flat = jax.lax.dynamic_update_slice(flat, sc1d, (btc * S,))
        return jnp.reshape(flat, shp)

    return (merge(trend, sc_t), merge(seasonal, sc_s), merge(resid, sc_r))
